# Initial kernel scaffold; baseline (speedup 1.0000x reference)
import functools

import jax
import jax.numpy as jnp
from jax import lax
from jax.experimental import pallas as pl
from jax.experimental.pallas import tpu as pltpu
from jax.experimental.pallas import tpu_sc as plsc

NUM_F = 9
VOCAB = 100
HIDDEN = 128
NWORKERS = 32
CHUNK = 32
TAB_WORDS = NUM_F * VOCAB * HIDDEN


def _body(rows_per_worker, x_hbm, tab_hbm, out_hbm, tab_v, xbuf, outbuf):
    wid = lax.axis_index("s") * 2 + lax.axis_index("c")
    base_row = wid * rows_per_worker

    pltpu.sync_copy(tab_hbm, tab_v)

    iota = lax.iota(jnp.int32, 16)
    num_chunks = rows_per_worker // CHUNK

    def chunk_body(c, _):
        row0 = base_row + c * CHUNK
        pltpu.sync_copy(x_hbm.at[pl.ds(row0 * NUM_F, CHUNK * NUM_F)], xbuf)

        for g in range(CHUNK // 16):
            fbases = []
            for i in range(NUM_F):
                col = plsc.load_gather(xbuf, [iota * NUM_F + (g * 16 * NUM_F + i)])
                fbases.append(col * HIDDEN + i * (VOCAB * HIDDEN))

            def row_body(r, _):
                rsplat = jnp.full((16,), 0, jnp.int32) + r
                bases = [fb.at[rsplat].get(mode="promise_in_bounds")
                         for fb in fbases]
                obase = (g * 16 + r) * HIDDEN
                for jb in range(HIDDEN // 16):
                    jv = jb * 16 + iota
                    acc = plsc.load_gather(tab_v, [bases[0] + jv])
                    for i in range(1, NUM_F):
                        acc = acc + plsc.load_gather(tab_v, [bases[i] + jv])
                    outbuf[pl.ds(obase + jb * 16, 16)] = acc
                return 0

            lax.fori_loop(0, 16, row_body, 0)

        pltpu.sync_copy(outbuf, out_hbm.at[pl.ds(row0 * HIDDEN, CHUNK * HIDDEN)])
        return 0

    lax.fori_loop(0, num_chunks, chunk_body, 0)


def kernel(x, tables):
    n = x.shape[0]
    npad = ((n + NWORKERS * CHUNK - 1) // (NWORKERS * CHUNK)) * (NWORKERS * CHUNK)
    rows_per_worker = npad // NWORKERS

    x_flat = jnp.pad(x.astype(jnp.int32), ((0, npad - n), (0, 0))).reshape(-1)
    tab_flat = tables.reshape(-1)

    mesh = plsc.VectorSubcoreMesh(core_axis_name="c", subcore_axis_name="s")
    run = pl.kernel(
        functools.partial(_body, rows_per_worker),
        out_type=jax.ShapeDtypeStruct((npad * HIDDEN,), jnp.float32),
        mesh=mesh,
        scratch_types=[
            pltpu.VMEM((TAB_WORDS,), jnp.float32),
            pltpu.VMEM((CHUNK * NUM_F,), jnp.int32),
            pltpu.VMEM((CHUNK * HIDDEN,), jnp.float32),
        ],
    )
    out_flat = run(x_flat, tab_flat)
    return out_flat.reshape(npad, HIDDEN)[:n]


# speedup vs baseline: 3.1966x; 3.1966x over previous
import functools

import jax
import jax.numpy as jnp
from jax import lax
from jax.experimental import pallas as pl
from jax.experimental.pallas import tpu as pltpu
from jax.experimental.pallas import tpu_sc as plsc

NUM_F = 9
VOCAB = 100
HIDDEN = 128
NWORKERS = 32
CHUNK = 32
TAB_WORDS = NUM_F * VOCAB * HIDDEN


def _body(rows_per_worker, x_hbm, tab_hbm, out_hbm, tab_v, xbuf, outbuf):
    wid = lax.axis_index("s") * 2 + lax.axis_index("c")
    base_row = wid * rows_per_worker

    pltpu.sync_copy(tab_hbm, tab_v)

    iota = lax.iota(jnp.int32, 16)
    num_chunks = rows_per_worker // CHUNK

    def chunk_body(c, _):
        row0 = base_row + c * CHUNK
        pltpu.sync_copy(x_hbm.at[pl.ds(row0 * NUM_F, CHUNK * NUM_F)], xbuf)

        for g in range(CHUNK // 16):
            fbases = []
            for i in range(NUM_F):
                col = plsc.load_gather(xbuf, [iota * NUM_F + (g * 16 * NUM_F + i)])
                fbases.append(col * HIDDEN + i * (VOCAB * HIDDEN))

            def row_body(r, _):
                rsplat = jnp.full((16,), 0, jnp.int32) + r
                bases = [fb.at[rsplat].get(mode="promise_in_bounds")
                         for fb in fbases]
                obase = (g * 16 + r) * HIDDEN
                for jb in range(HIDDEN // 16):
                    jv = jb * 16 + iota
                    acc = plsc.load_gather(tab_v, [bases[0] + jv])
                    for i in range(1, NUM_F):
                        acc = acc + plsc.load_gather(tab_v, [bases[i] + jv])
                    outbuf[pl.ds(obase + jb * 16, 16)] = acc
                return 0

            lax.fori_loop(0, 16, row_body, 0)

        pltpu.sync_copy(outbuf, out_hbm.at[pl.ds(row0 * HIDDEN, CHUNK * HIDDEN)])
        return 0

    lax.fori_loop(0, num_chunks, chunk_body, 0)


def kernel(x, tables):
    n = x.shape[0]
    npad = ((n + NWORKERS * CHUNK - 1) // (NWORKERS * CHUNK)) * (NWORKERS * CHUNK)
    rows_per_worker = npad // NWORKERS

    x_flat = jnp.pad(x.astype(jnp.int32), ((0, npad - n), (0, 0))).reshape(-1)
    tab_flat = tables.reshape(-1)

    mesh = plsc.VectorSubcoreMesh(
        core_axis_name="c", subcore_axis_name="s", num_cores=2, num_subcores=16
    )
    run = pl.kernel(
        functools.partial(_body, rows_per_worker),
        out_type=jax.ShapeDtypeStruct((npad * HIDDEN,), jnp.float32),
        mesh=mesh,
        compiler_params=pltpu.CompilerParams(needs_layout_passes=False),
        scratch_types=[
            pltpu.VMEM((TAB_WORDS,), jnp.float32),
            pltpu.VMEM((CHUNK * NUM_F,), jnp.int32),
            pltpu.VMEM((CHUNK * HIDDEN,), jnp.float32),
        ],
    )
    out_flat = run(x_flat, tab_flat)
    return out_flat.reshape(npad, HIDDEN)[:n]


# speedup vs baseline: 3.6470x; 1.1409x over previous
import functools

import jax
import jax.numpy as jnp
from jax import lax
from jax.experimental import pallas as pl
from jax.experimental.pallas import tpu as pltpu
from jax.experimental.pallas import tpu_sc as plsc

NUM_F = 9
VOCAB = 100
HIDDEN = 128
NWORKERS = 32
CHUNK = 32
XW = CHUNK * NUM_F
OW = CHUNK * HIDDEN
TAB_WORDS = NUM_F * VOCAB * HIDDEN


def _body(rows_per_worker, x_hbm, tab_hbm, out_hbm,
          tab_v, xb0, xb1, ob0, ob1, sx0, sx1, so0, so1):
    wid = lax.axis_index("s") * 2 + lax.axis_index("c")
    base_row = wid * rows_per_worker

    pltpu.sync_copy(tab_hbm, tab_v)

    iota = lax.iota(jnp.int32, 16)
    num_chunks = rows_per_worker // CHUNK
    npairs = num_chunks // 2

    def x_slice(c):
        return x_hbm.at[pl.ds((base_row + c * CHUNK) * NUM_F, XW)]

    def o_slice(c):
        return out_hbm.at[pl.ds((base_row + c * CHUNK) * HIDDEN, OW)]

    def compute(xbuf, outbuf):
        for g in range(CHUNK // 16):
            fbases = []
            for i in range(NUM_F):
                col = plsc.load_gather(xbuf, [iota * NUM_F + (g * 16 * NUM_F + i)])
                fbases.append(col * HIDDEN + i * (VOCAB * HIDDEN))

            def row_body(r, _):
                rsplat = jnp.full((16,), 0, jnp.int32) + r
                vidx = [fb.at[rsplat].get(mode="promise_in_bounds") + iota
                        for fb in fbases]
                obase = (g * 16 + r) * HIDDEN
                for jb in range(HIDDEN // 16):
                    tv = tab_v.at[pl.ds(jb * 16, TAB_WORDS - jb * 16)]
                    acc = plsc.load_gather(tv, [vidx[0]])
                    for i in range(1, NUM_F):
                        acc = acc + plsc.load_gather(tv, [vidx[i]])
                    outbuf[pl.ds(obase + jb * 16, 16)] = acc
                return 0

            lax.fori_loop(0, 16, row_body, 0)

    pltpu.async_copy(x_slice(0), xb0, sx0)

    def pair_body(p, _):
        ca = 2 * p
        cb = ca + 1
        pltpu.async_copy(x_slice(cb), xb1, sx1)

        pltpu.make_async_copy(x_slice(ca), xb0, sx0).wait()

        @pl.when(p > 0)
        def _():
            pltpu.make_async_copy(ob0, o_slice(ca), so0).wait()

        compute(xb0, ob0)
        pltpu.async_copy(ob0, o_slice(ca), so0)

        @pl.when(p < npairs - 1)
        def _():
            pltpu.async_copy(x_slice(ca + 2), xb0, sx0)

        pltpu.make_async_copy(x_slice(cb), xb1, sx1).wait()

        @pl.when(p > 0)
        def _():
            pltpu.make_async_copy(ob1, o_slice(cb), so1).wait()

        compute(xb1, ob1)
        pltpu.async_copy(ob1, o_slice(cb), so1)
        return 0

    lax.fori_loop(0, npairs, pair_body, 0)

    pltpu.make_async_copy(ob0, o_slice(num_chunks - 2), so0).wait()
    pltpu.make_async_copy(ob1, o_slice(num_chunks - 1), so1).wait()


def kernel(x, tables):
    n = x.shape[0]
    block = NWORKERS * CHUNK * 2
    npad = ((n + block - 1) // block) * block
    rows_per_worker = npad // NWORKERS

    x_flat = jnp.pad(x.astype(jnp.int32), ((0, npad - n), (0, 0))).reshape(-1)
    tab_flat = tables.reshape(-1)

    mesh = plsc.VectorSubcoreMesh(
        core_axis_name="c", subcore_axis_name="s", num_cores=2, num_subcores=16
    )
    run = pl.kernel(
        functools.partial(_body, rows_per_worker),
        out_type=jax.ShapeDtypeStruct((npad * HIDDEN,), jnp.float32),
        mesh=mesh,
        compiler_params=pltpu.CompilerParams(needs_layout_passes=False),
        scratch_types=[
            pltpu.VMEM((TAB_WORDS,), jnp.float32),
            pltpu.VMEM((XW,), jnp.int32),
            pltpu.VMEM((XW,), jnp.int32),
            pltpu.VMEM((OW,), jnp.float32),
            pltpu.VMEM((OW,), jnp.float32),
            pltpu.SemaphoreType.DMA,
            pltpu.SemaphoreType.DMA,
            pltpu.SemaphoreType.DMA,
            pltpu.SemaphoreType.DMA,
        ],
    )
    out_flat = run(x_flat, tab_flat)
    return out_flat.reshape(npad, HIDDEN)[:n]


# speedup vs baseline: 3.6671x; 1.0055x over previous
import functools

import jax
import jax.numpy as jnp
from jax import lax
from jax.experimental import pallas as pl
from jax.experimental.pallas import tpu as pltpu
from jax.experimental.pallas import tpu_sc as plsc

NUM_F = 9
VOCAB = 100
HIDDEN = 128
NWORKERS = 32
CHUNK = 32
XW = CHUNK * NUM_F
OW = CHUNK * HIDDEN
TAB_WORDS = NUM_F * VOCAB * HIDDEN


def _tree_sum9(vals):
    s01 = vals[0] + vals[1]
    s23 = vals[2] + vals[3]
    s45 = vals[4] + vals[5]
    s67 = vals[6] + vals[7]
    a = s01 + s23
    b = s45 + s67
    return (a + b) + vals[8]


def _body(rows_per_worker, x_hbm, tab_hbm, out_hbm,
          tab_v, xb0, xb1, ob0, ob1, sx0, sx1, so0, so1):
    wid = lax.axis_index("s") * 2 + lax.axis_index("c")
    base_row = wid * rows_per_worker

    pltpu.sync_copy(tab_hbm, tab_v)

    num_chunks = rows_per_worker // CHUNK
    npairs = num_chunks // 2

    def x_slice(c):
        return x_hbm.at[pl.ds((base_row + c * CHUNK) * NUM_F, XW)]

    def o_slice(c):
        return out_hbm.at[pl.ds((base_row + c * CHUNK) * HIDDEN, OW)]

    def compute(xbuf, outbuf):
        def row_body(r, _):
            xv = xbuf[pl.ds(r * NUM_F, 16)]
            bases = [xv[i] * HIDDEN + i * (VOCAB * HIDDEN)
                     for i in range(NUM_F)]
            obase = r * HIDDEN
            for jb in range(HIDDEN // 16):
                loads = [tab_v[pl.ds(bases[i] + jb * 16, 16)]
                         for i in range(NUM_F)]
                outbuf[pl.ds(obase + jb * 16, 16)] = _tree_sum9(loads)
            return 0

        lax.fori_loop(0, CHUNK, row_body, 0)

    pltpu.async_copy(x_slice(0), xb0.at[pl.ds(0, XW)], sx0)

    def pair_body(p, _):
        ca = 2 * p
        cb = ca + 1
        pltpu.async_copy(x_slice(cb), xb1.at[pl.ds(0, XW)], sx1)

        pltpu.make_async_copy(x_slice(ca), xb0.at[pl.ds(0, XW)], sx0).wait()

        @pl.when(p > 0)
        def _():
            pltpu.make_async_copy(ob0, o_slice(ca), so0).wait()

        compute(xb0, ob0)
        pltpu.async_copy(ob0, o_slice(ca), so0)

        @pl.when(p < npairs - 1)
        def _():
            pltpu.async_copy(x_slice(ca + 2), xb0.at[pl.ds(0, XW)], sx0)

        pltpu.make_async_copy(x_slice(cb), xb1.at[pl.ds(0, XW)], sx1).wait()

        @pl.when(p > 0)
        def _():
            pltpu.make_async_copy(ob1, o_slice(cb), so1).wait()

        compute(xb1, ob1)
        pltpu.async_copy(ob1, o_slice(cb), so1)
        return 0

    lax.fori_loop(0, npairs, pair_body, 0)

    pltpu.make_async_copy(ob0, o_slice(num_chunks - 2), so0).wait()
    pltpu.make_async_copy(ob1, o_slice(num_chunks - 1), so1).wait()


def kernel(x, tables):
    n = x.shape[0]
    block = NWORKERS * CHUNK * 2
    npad = ((n + block - 1) // block) * block
    rows_per_worker = npad // NWORKERS

    x_flat = jnp.pad(x.astype(jnp.int32), ((0, npad - n), (0, 0))).reshape(-1)
    tab_flat = tables.reshape(-1)

    mesh = plsc.VectorSubcoreMesh(
        core_axis_name="c", subcore_axis_name="s", num_cores=2, num_subcores=16
    )
    run = pl.kernel(
        functools.partial(_body, rows_per_worker),
        out_type=jax.ShapeDtypeStruct((npad * HIDDEN,), jnp.float32),
        mesh=mesh,
        compiler_params=pltpu.CompilerParams(needs_layout_passes=False),
        scratch_types=[
            pltpu.VMEM((TAB_WORDS,), jnp.float32),
            pltpu.VMEM((XW + 16,), jnp.int32),
            pltpu.VMEM((XW + 16,), jnp.int32),
            pltpu.VMEM((OW,), jnp.float32),
            pltpu.VMEM((OW,), jnp.float32),
            pltpu.SemaphoreType.DMA,
            pltpu.SemaphoreType.DMA,
            pltpu.SemaphoreType.DMA,
            pltpu.SemaphoreType.DMA,
        ],
    )
    out_flat = run(x_flat, tab_flat)
    return out_flat.reshape(npad, HIDDEN)[:n]


# speedup vs baseline: 4.1860x; 1.1415x over previous
import functools

import jax
import jax.numpy as jnp
from jax import lax
from jax.experimental import pallas as pl
from jax.experimental.pallas import tpu as pltpu
from jax.experimental.pallas import tpu_sc as plsc

NUM_F = 9
VOCAB = 100
HIDDEN = 128
NWORKERS = 32
CHUNK = 32
XW = CHUNK * NUM_F
OW = CHUNK * HIDDEN
TAB_WORDS = NUM_F * VOCAB * HIDDEN


def _tree_sum9(vals):
    s01 = vals[0] + vals[1]
    s23 = vals[2] + vals[3]
    s45 = vals[4] + vals[5]
    s67 = vals[6] + vals[7]
    a = s01 + s23
    b = s45 + s67
    return (a + b) + vals[8]


def _body(rows_per_worker, last_rows, x_hbm, tab_hbm, out_hbm,
          tab_v, xb0, xb1, ob0, ob1, sx0, sx1, so0, so1):
    wid = lax.axis_index("s") * 2 + lax.axis_index("c")
    base_row = wid * rows_per_worker

    pltpu.sync_copy(tab_hbm, tab_v)

    my_rows = jnp.where(wid == NWORKERS - 1, last_rows, rows_per_worker)
    num_chunks = my_rows // CHUNK
    npairs = num_chunks // 2

    def x_slice(c):
        return x_hbm.at[pl.ds((base_row + c * CHUNK) * NUM_F, XW)]

    def o_slice(c):
        return out_hbm.at[pl.ds((base_row + c * CHUNK) * HIDDEN, OW)]

    def compute(xbuf, outbuf):
        def row_body(r, _):
            xv = xbuf[pl.ds(r * NUM_F, 16)]
            bases = [xv[i] * HIDDEN + i * (VOCAB * HIDDEN)
                     for i in range(NUM_F)]
            obase = r * HIDDEN
            for jb in range(HIDDEN // 16):
                loads = [tab_v[pl.ds(bases[i] + jb * 16, 16)]
                         for i in range(NUM_F)]
                outbuf[pl.ds(obase + jb * 16, 16)] = _tree_sum9(loads)
            return 0

        lax.fori_loop(0, CHUNK, row_body, 0)

    pltpu.async_copy(x_slice(0), xb0.at[pl.ds(0, XW)], sx0)

    def pair_body(p, _):
        ca = 2 * p
        cb = ca + 1
        pltpu.async_copy(x_slice(cb), xb1.at[pl.ds(0, XW)], sx1)

        pltpu.make_async_copy(x_slice(ca), xb0.at[pl.ds(0, XW)], sx0).wait()

        @pl.when(p > 0)
        def _():
            pltpu.make_async_copy(ob0, o_slice(ca), so0).wait()

        compute(xb0, ob0)
        pltpu.async_copy(ob0, o_slice(ca), so0)

        @pl.when(p < npairs - 1)
        def _():
            pltpu.async_copy(x_slice(ca + 2), xb0.at[pl.ds(0, XW)], sx0)

        pltpu.make_async_copy(x_slice(cb), xb1.at[pl.ds(0, XW)], sx1).wait()

        @pl.when(p > 0)
        def _():
            pltpu.make_async_copy(ob1, o_slice(cb), so1).wait()

        compute(xb1, ob1)
        pltpu.async_copy(ob1, o_slice(cb), so1)
        return 0

    lax.fori_loop(0, npairs, pair_body, 0)

    @pl.when(npairs > 0)
    def _():
        pltpu.make_async_copy(ob0, o_slice(0), so0).wait()
        pltpu.make_async_copy(ob1, o_slice(0), so1).wait()

    @pl.when(num_chunks % 2 == 1)
    def _():
        c = num_chunks - 1
        pltpu.sync_copy(x_slice(c), xb0.at[pl.ds(0, XW)])
        compute(xb0, ob0)
        pltpu.sync_copy(ob0, o_slice(c))


def kernel(x, tables):
    n = x.shape[0]
    n32 = ((n + CHUNK - 1) // CHUNK) * CHUNK
    if n32 != n:
        x = jnp.pad(x, ((0, n32 - n), (0, 0)))
    rows_per_worker = ((n32 + NWORKERS * CHUNK - 1) // (NWORKERS * CHUNK)) * CHUNK
    last_rows = n32 - (NWORKERS - 1) * rows_per_worker
    assert last_rows >= 0

    x_flat = x.astype(jnp.int32).reshape(-1)
    tab_flat = tables.reshape(-1)

    mesh = plsc.VectorSubcoreMesh(
        core_axis_name="c", subcore_axis_name="s", num_cores=2, num_subcores=16
    )
    run = pl.kernel(
        functools.partial(_body, rows_per_worker, last_rows),
        out_type=jax.ShapeDtypeStruct((n32 * HIDDEN,), jnp.float32),
        mesh=mesh,
        compiler_params=pltpu.CompilerParams(needs_layout_passes=False),
        scratch_types=[
            pltpu.VMEM((TAB_WORDS,), jnp.float32),
            pltpu.VMEM((XW + 16,), jnp.int32),
            pltpu.VMEM((XW + 16,), jnp.int32),
            pltpu.VMEM((OW,), jnp.float32),
            pltpu.VMEM((OW,), jnp.float32),
            pltpu.SemaphoreType.DMA,
            pltpu.SemaphoreType.DMA,
            pltpu.SemaphoreType.DMA,
            pltpu.SemaphoreType.DMA,
        ],
    )
    out_flat = run(x_flat, tab_flat)
    out = out_flat.reshape(n32, HIDDEN)
    return out[:n] if n32 != n else out


# speedup vs baseline: 7.7640x; 1.8547x over previous
import functools

import jax
import jax.numpy as jnp
from jax import lax
from jax.experimental import pallas as pl
from jax.experimental.pallas import tpu as pltpu
from jax.experimental.pallas import tpu_sc as plsc

NUM_F = 9
VOCAB = 100
HIDDEN = 128
NWORKERS = 32
CHUNK = 32
XW = CHUNK * NUM_F
OW = CHUNK * HIDDEN
TAB_WORDS = NUM_F * VOCAB * HIDDEN


def _tree_sum9(vals):
    s01 = vals[0] + vals[1]
    s23 = vals[2] + vals[3]
    s45 = vals[4] + vals[5]
    s67 = vals[6] + vals[7]
    a = s01 + s23
    b = s45 + s67
    return (a + b) + vals[8]


def _body(rows_per_worker, last_rows, x_hbm, tab_hbm, out_hbm,
          tab_v, xb0, xb1, ob0, ob1, sx0, sx1, so0, so1):
    wid = lax.axis_index("s") * 2 + lax.axis_index("c")
    base_row = wid * rows_per_worker

    pltpu.sync_copy(tab_hbm, tab_v)

    my_rows = jnp.where(wid == NWORKERS - 1, last_rows, rows_per_worker)
    num_chunks = my_rows // CHUNK
    npairs = num_chunks // 2

    def x_slice(c):
        return x_hbm.at[pl.ds((base_row + c * CHUNK) * NUM_F, XW)]

    def o_slice(c):
        return out_hbm.at[pl.ds((base_row + c * CHUNK) * HIDDEN, OW)]

    def compute(xbuf, outbuf):
        @plsc.parallel_loop(0, CHUNK, 1, unroll=2)
        def row_body(r):
            xv = xbuf[pl.ds(r * NUM_F, 16)]
            bases = [xv[i] * HIDDEN + i * (VOCAB * HIDDEN)
                     for i in range(NUM_F)]
            obase = r * HIDDEN
            for jb in range(HIDDEN // 16):
                loads = [tab_v[pl.ds(bases[i] + jb * 16, 16)]
                         for i in range(NUM_F)]
                outbuf[pl.ds(obase + jb * 16, 16)] = _tree_sum9(loads)

    pltpu.async_copy(x_slice(0), xb0.at[pl.ds(0, XW)], sx0)

    def pair_body(p, _):
        ca = 2 * p
        cb = ca + 1
        pltpu.async_copy(x_slice(cb), xb1.at[pl.ds(0, XW)], sx1)

        pltpu.make_async_copy(x_slice(ca), xb0.at[pl.ds(0, XW)], sx0).wait()

        @pl.when(p > 0)
        def _():
            pltpu.make_async_copy(ob0, o_slice(ca), so0).wait()

        compute(xb0, ob0)
        pltpu.async_copy(ob0, o_slice(ca), so0)

        @pl.when(p < npairs - 1)
        def _():
            pltpu.async_copy(x_slice(ca + 2), xb0.at[pl.ds(0, XW)], sx0)

        pltpu.make_async_copy(x_slice(cb), xb1.at[pl.ds(0, XW)], sx1).wait()

        @pl.when(p > 0)
        def _():
            pltpu.make_async_copy(ob1, o_slice(cb), so1).wait()

        compute(xb1, ob1)
        pltpu.async_copy(ob1, o_slice(cb), so1)
        return 0

    lax.fori_loop(0, npairs, pair_body, 0)

    @pl.when(npairs > 0)
    def _():
        pltpu.make_async_copy(ob0, o_slice(0), so0).wait()
        pltpu.make_async_copy(ob1, o_slice(0), so1).wait()

    @pl.when(num_chunks % 2 == 1)
    def _():
        c = num_chunks - 1
        pltpu.sync_copy(x_slice(c), xb0.at[pl.ds(0, XW)])
        compute(xb0, ob0)
        pltpu.sync_copy(ob0, o_slice(c))


def kernel(x, tables):
    n = x.shape[0]
    n32 = ((n + CHUNK - 1) // CHUNK) * CHUNK
    if n32 != n:
        x = jnp.pad(x, ((0, n32 - n), (0, 0)))
    rows_per_worker = ((n32 + NWORKERS * CHUNK - 1) // (NWORKERS * CHUNK)) * CHUNK
    last_rows = n32 - (NWORKERS - 1) * rows_per_worker
    assert last_rows >= 0

    x_flat = x.astype(jnp.int32).reshape(-1)
    tab_flat = tables.reshape(-1)

    mesh = plsc.VectorSubcoreMesh(
        core_axis_name="c", subcore_axis_name="s", num_cores=2, num_subcores=16
    )
    run = pl.kernel(
        functools.partial(_body, rows_per_worker, last_rows),
        out_type=jax.ShapeDtypeStruct((n32 * HIDDEN,), jnp.float32),
        mesh=mesh,
        compiler_params=pltpu.CompilerParams(needs_layout_passes=False),
        scratch_types=[
            pltpu.VMEM((TAB_WORDS,), jnp.float32),
            pltpu.VMEM((XW + 16,), jnp.int32),
            pltpu.VMEM((XW + 16,), jnp.int32),
            pltpu.VMEM((OW,), jnp.float32),
            pltpu.VMEM((OW,), jnp.float32),
            pltpu.SemaphoreType.DMA,
            pltpu.SemaphoreType.DMA,
            pltpu.SemaphoreType.DMA,
            pltpu.SemaphoreType.DMA,
        ],
    )
    out_flat = run(x_flat, tab_flat)
    out = out_flat.reshape(n32, HIDDEN)
    return out[:n] if n32 != n else out


# speedup vs baseline: 7.9636x; 1.0257x over previous
import functools

import jax
import jax.numpy as jnp
from jax import lax
from jax.experimental import pallas as pl
from jax.experimental.pallas import tpu as pltpu
from jax.experimental.pallas import tpu_sc as plsc

NUM_F = 9
VOCAB = 100
HIDDEN = 128
NWORKERS = 32
CHUNK = 32
XW = CHUNK * NUM_F
OW = CHUNK * HIDDEN
TAB_WORDS = NUM_F * VOCAB * HIDDEN


def _tree_sum9(vals):
    s01 = vals[0] + vals[1]
    s23 = vals[2] + vals[3]
    s45 = vals[4] + vals[5]
    s67 = vals[6] + vals[7]
    a = s01 + s23
    b = s45 + s67
    return (a + b) + vals[8]


def _interleave_cols(tab):
    r, c = tab.shape
    t = tab.reshape(r, c // 32, 2, 16).transpose(0, 1, 3, 2)
    return t.reshape(r, c)


def _body(rows_per_worker, last_rows, x_hbm, tab_hbm, out_hbm,
          tab_v, xb0, xb1, ob0, ob1, sx0, sx1, so0, so1):
    wid = lax.axis_index("s") * 2 + lax.axis_index("c")
    base_row = wid * rows_per_worker

    pltpu.sync_copy(tab_hbm, tab_v)

    my_rows = jnp.where(wid == NWORKERS - 1, last_rows, rows_per_worker)
    num_chunks = my_rows // CHUNK
    npairs = num_chunks // 2

    def x_slice(c):
        return x_hbm.at[pl.ds((base_row + c * CHUNK) * NUM_F, XW)]

    def o_slice(c):
        return out_hbm.at[pl.ds((base_row + c * CHUNK) * HIDDEN, OW)]

    def compute(xbuf, outbuf):
        @plsc.parallel_loop(0, CHUNK, 1, unroll=2)
        def row_body(r):
            xv = xbuf[pl.ds(r * NUM_F, 16)]
            bases = [xv[i] * HIDDEN + i * (VOCAB * HIDDEN)
                     for i in range(NUM_F)]
            obase = r * HIDDEN
            for sb in range(HIDDEN // 32):
                loads = [tab_v[pl.ds(bases[i] + sb * 32, 32)]
                         for i in range(NUM_F)]
                lo, hi = plsc.unpack(_tree_sum9(loads),
                                     format=plsc.PackFormat.INTERLEAVED)
                outbuf[pl.ds(obase + sb * 32, 16)] = lo
                outbuf[pl.ds(obase + sb * 32 + 16, 16)] = hi

    pltpu.async_copy(x_slice(0), xb0.at[pl.ds(0, XW)], sx0)

    def pair_body(p, _):
        ca = 2 * p
        cb = ca + 1
        pltpu.async_copy(x_slice(cb), xb1.at[pl.ds(0, XW)], sx1)

        pltpu.make_async_copy(x_slice(ca), xb0.at[pl.ds(0, XW)], sx0).wait()

        @pl.when(p > 0)
        def _():
            pltpu.make_async_copy(ob0, o_slice(ca), so0).wait()

        compute(xb0, ob0)
        pltpu.async_copy(ob0, o_slice(ca), so0)

        @pl.when(p < npairs - 1)
        def _():
            pltpu.async_copy(x_slice(ca + 2), xb0.at[pl.ds(0, XW)], sx0)

        pltpu.make_async_copy(x_slice(cb), xb1.at[pl.ds(0, XW)], sx1).wait()

        @pl.when(p > 0)
        def _():
            pltpu.make_async_copy(ob1, o_slice(cb), so1).wait()

        compute(xb1, ob1)
        pltpu.async_copy(ob1, o_slice(cb), so1)
        return 0

    lax.fori_loop(0, npairs, pair_body, 0)

    @pl.when(npairs > 0)
    def _():
        pltpu.make_async_copy(ob0, o_slice(0), so0).wait()
        pltpu.make_async_copy(ob1, o_slice(0), so1).wait()

    @pl.when(num_chunks % 2 == 1)
    def _():
        c = num_chunks - 1
        pltpu.sync_copy(x_slice(c), xb0.at[pl.ds(0, XW)])
        compute(xb0, ob0)
        pltpu.sync_copy(ob0, o_slice(c))


def kernel(x, tables):
    n = x.shape[0]
    n32 = ((n + CHUNK - 1) // CHUNK) * CHUNK
    if n32 != n:
        x = jnp.pad(x, ((0, n32 - n), (0, 0)))
    rows_per_worker = ((n32 + NWORKERS * CHUNK - 1) // (NWORKERS * CHUNK)) * CHUNK
    last_rows = n32 - (NWORKERS - 1) * rows_per_worker
    assert last_rows >= 0

    x_flat = x.astype(jnp.int32).reshape(-1)
    tab16 = _interleave_cols(
        tables.reshape(NUM_F * VOCAB, HIDDEN).astype(jnp.bfloat16))
    tab_flat = tab16.reshape(-1)

    mesh = plsc.VectorSubcoreMesh(
        core_axis_name="c", subcore_axis_name="s", num_cores=2, num_subcores=16
    )
    run = pl.kernel(
        functools.partial(_body, rows_per_worker, last_rows),
        out_type=jax.ShapeDtypeStruct((n32 * HIDDEN,), jnp.float32),
        mesh=mesh,
        compiler_params=pltpu.CompilerParams(needs_layout_passes=False),
        scratch_types=[
            pltpu.VMEM((TAB_WORDS,), jnp.bfloat16),
            pltpu.VMEM((XW + 16,), jnp.int32),
            pltpu.VMEM((XW + 16,), jnp.int32),
            pltpu.VMEM((OW,), jnp.float32),
            pltpu.VMEM((OW,), jnp.float32),
            pltpu.SemaphoreType.DMA,
            pltpu.SemaphoreType.DMA,
            pltpu.SemaphoreType.DMA,
            pltpu.SemaphoreType.DMA,
        ],
    )
    out_flat = run(x_flat, tab_flat)
    out = out_flat.reshape(n32, HIDDEN)
    return out[:n] if n32 != n else out


# speedup vs baseline: 10.8905x; 1.3675x over previous
import functools

import jax
import jax.numpy as jnp
from jax import lax
from jax.experimental import pallas as pl
from jax.experimental.pallas import tpu as pltpu
from jax.experimental.pallas import tpu_sc as plsc

NUM_F = 9
VOCAB = 100
HIDDEN = 128
NWORKERS = 32
CHUNK = 32
XW = CHUNK * NUM_F
OW = CHUNK * HIDDEN
TAB_WORDS = NUM_F * VOCAB * HIDDEN


def _tree_sum9(vals):
    s01 = vals[0] + vals[1]
    s23 = vals[2] + vals[3]
    s45 = vals[4] + vals[5]
    s67 = vals[6] + vals[7]
    a = s01 + s23
    b = s45 + s67
    return (a + b) + vals[8]


def _pack_table(tab):
    r, c = tab.shape
    t = tab.astype(jnp.bfloat16).reshape(r, c // 32, 2, 16).transpose(0, 1, 3, 2)
    return lax.bitcast_convert_type(t, jnp.int32).reshape(-1)


def _body(rows_per_worker, last_rows, x_hbm, tab_hbm, out_hbm,
          tab_v, xb0, xb1, ob0, ob1, sx0, sx1, so0, so1):
    wid = lax.axis_index("s") * 2 + lax.axis_index("c")
    base_row = wid * rows_per_worker

    pltpu.sync_copy(tab_hbm, tab_v)

    my_rows = jnp.where(wid == NWORKERS - 1, last_rows, rows_per_worker)
    num_chunks = my_rows // CHUNK
    npairs = num_chunks // 2

    def x_slice(c):
        return x_hbm.at[pl.ds((base_row + c * CHUNK) * NUM_F, XW)]

    def o_slice(c):
        return out_hbm.at[pl.ds((base_row + c * CHUNK) * HIDDEN, OW)]

    def compute(xbuf, outbuf):
        @plsc.parallel_loop(0, CHUNK, 1, unroll=2)
        def row_body(r):
            xv = xbuf[pl.ds(r * NUM_F, 16)]
            bases = [xv[i] * (HIDDEN // 2) + i * (VOCAB * HIDDEN // 2)
                     for i in range(NUM_F)]
            obase = r * HIDDEN
            for sb in range(HIDDEN // 32):
                loads = [plsc.bitcast(tab_v[pl.ds(bases[i] + sb * 16, 16)],
                                      jnp.bfloat16)
                         for i in range(NUM_F)]
                lo, hi = plsc.unpack(_tree_sum9(loads),
                                     format=plsc.PackFormat.INTERLEAVED)
                outbuf[pl.ds(obase + sb * 32, 16)] = lo
                outbuf[pl.ds(obase + sb * 32 + 16, 16)] = hi

    pltpu.async_copy(x_slice(0), xb0.at[pl.ds(0, XW)], sx0)

    def pair_body(p, _):
        ca = 2 * p
        cb = ca + 1
        pltpu.async_copy(x_slice(cb), xb1.at[pl.ds(0, XW)], sx1)

        pltpu.make_async_copy(x_slice(ca), xb0.at[pl.ds(0, XW)], sx0).wait()

        @pl.when(p > 0)
        def _():
            pltpu.make_async_copy(ob0, o_slice(ca), so0).wait()

        compute(xb0, ob0)
        pltpu.async_copy(ob0, o_slice(ca), so0)

        @pl.when(p < npairs - 1)
        def _():
            pltpu.async_copy(x_slice(ca + 2), xb0.at[pl.ds(0, XW)], sx0)

        pltpu.make_async_copy(x_slice(cb), xb1.at[pl.ds(0, XW)], sx1).wait()

        @pl.when(p > 0)
        def _():
            pltpu.make_async_copy(ob1, o_slice(cb), so1).wait()

        compute(xb1, ob1)
        pltpu.async_copy(ob1, o_slice(cb), so1)
        return 0

    lax.fori_loop(0, npairs, pair_body, 0)

    @pl.when(npairs > 0)
    def _():
        pltpu.make_async_copy(ob0, o_slice(0), so0).wait()
        pltpu.make_async_copy(ob1, o_slice(0), so1).wait()

    @pl.when(num_chunks % 2 == 1)
    def _():
        c = num_chunks - 1
        pltpu.sync_copy(x_slice(c), xb0.at[pl.ds(0, XW)])
        compute(xb0, ob0)
        pltpu.sync_copy(ob0, o_slice(c))


def kernel(x, tables):
    n = x.shape[0]
    n32 = ((n + CHUNK - 1) // CHUNK) * CHUNK
    if n32 != n:
        x = jnp.pad(x, ((0, n32 - n), (0, 0)))
    rows_per_worker = ((n32 + NWORKERS * CHUNK - 1) // (NWORKERS * CHUNK)) * CHUNK
    last_rows = n32 - (NWORKERS - 1) * rows_per_worker
    assert last_rows >= 0

    x_flat = x.astype(jnp.int32).reshape(-1)
    tab_flat = _pack_table(tables.reshape(NUM_F * VOCAB, HIDDEN))

    mesh = plsc.VectorSubcoreMesh(
        core_axis_name="c", subcore_axis_name="s", num_cores=2, num_subcores=16
    )
    run = pl.kernel(
        functools.partial(_body, rows_per_worker, last_rows),
        out_type=jax.ShapeDtypeStruct((n32 * HIDDEN,), jnp.float32),
        mesh=mesh,
        compiler_params=pltpu.CompilerParams(needs_layout_passes=False),
        scratch_types=[
            pltpu.VMEM((TAB_WORDS // 2,), jnp.int32),
            pltpu.VMEM((XW + 16,), jnp.int32),
            pltpu.VMEM((XW + 16,), jnp.int32),
            pltpu.VMEM((OW,), jnp.float32),
            pltpu.VMEM((OW,), jnp.float32),
            pltpu.SemaphoreType.DMA,
            pltpu.SemaphoreType.DMA,
            pltpu.SemaphoreType.DMA,
            pltpu.SemaphoreType.DMA,
        ],
    )
    out_flat = run(x_flat, tab_flat)
    out = out_flat.reshape(n32, HIDDEN)
    return out[:n] if n32 != n else out


# speedup vs baseline: 10.8981x; 1.0007x over previous
import functools

import jax
import jax.numpy as jnp
from jax import lax
from jax.experimental import pallas as pl
from jax.experimental.pallas import tpu as pltpu
from jax.experimental.pallas import tpu_sc as plsc

NUM_F = 9
VOCAB = 100
HIDDEN = 128
NWORKERS = 32
CHUNK = 32
XW = CHUNK * NUM_F
OW = CHUNK * HIDDEN
TAB_WORDS = NUM_F * VOCAB * HIDDEN


def _tree_sum9(vals):
    s01 = vals[0] + vals[1]
    s23 = vals[2] + vals[3]
    s45 = vals[4] + vals[5]
    s67 = vals[6] + vals[7]
    a = s01 + s23
    b = s45 + s67
    return (a + b) + vals[8]


def _pack_table(tab):
    r, c = tab.shape
    t = tab.astype(jnp.bfloat16).reshape(r, c // 32, 2, 16).transpose(0, 1, 3, 2)
    return lax.bitcast_convert_type(t, jnp.int32).reshape(-1)


def _body(rows_per_worker, last_rows, x_hbm, tab_hbm, out_hbm,
          tab_v, xb0, xb1, ob0, ob1, sx0, sx1, so0, so1):
    wid = lax.axis_index("s") * 2 + lax.axis_index("c")
    base_row = wid * rows_per_worker

    pltpu.sync_copy(tab_hbm, tab_v)

    my_rows = jnp.where(wid == NWORKERS - 1, last_rows, rows_per_worker)
    num_chunks = my_rows // CHUNK
    npairs = num_chunks // 2

    def x_slice(c):
        return x_hbm.at[pl.ds((base_row + c * CHUNK) * NUM_F, XW)]

    def o_slice(c):
        return out_hbm.at[pl.ds(base_row + c * CHUNK, CHUNK), :]

    def compute(xbuf, outbuf):
        @plsc.parallel_loop(0, CHUNK, 1, unroll=2)
        def row_body(r):
            xv = xbuf[pl.ds(r * NUM_F, 16)]
            bases = [xv[i] * (HIDDEN // 2) + i * (VOCAB * HIDDEN // 2)
                     for i in range(NUM_F)]
            for sb in range(HIDDEN // 32):
                loads = [plsc.bitcast(tab_v[pl.ds(bases[i] + sb * 16, 16)],
                                      jnp.bfloat16)
                         for i in range(NUM_F)]
                lo, hi = plsc.unpack(_tree_sum9(loads),
                                     format=plsc.PackFormat.INTERLEAVED)
                outbuf[r, pl.ds(sb * 32, 16)] = lo
                outbuf[r, pl.ds(sb * 32 + 16, 16)] = hi

    pltpu.async_copy(x_slice(0), xb0.at[pl.ds(0, XW)], sx0)

    def pair_body(p, _):
        ca = 2 * p
        cb = ca + 1
        pltpu.async_copy(x_slice(cb), xb1.at[pl.ds(0, XW)], sx1)

        pltpu.make_async_copy(x_slice(ca), xb0.at[pl.ds(0, XW)], sx0).wait()

        @pl.when(p > 0)
        def _():
            pltpu.make_async_copy(ob0, o_slice(ca), so0).wait()

        compute(xb0, ob0)
        pltpu.async_copy(ob0, o_slice(ca), so0)

        @pl.when(p < npairs - 1)
        def _():
            pltpu.async_copy(x_slice(ca + 2), xb0.at[pl.ds(0, XW)], sx0)

        pltpu.make_async_copy(x_slice(cb), xb1.at[pl.ds(0, XW)], sx1).wait()

        @pl.when(p > 0)
        def _():
            pltpu.make_async_copy(ob1, o_slice(cb), so1).wait()

        compute(xb1, ob1)
        pltpu.async_copy(ob1, o_slice(cb), so1)
        return 0

    lax.fori_loop(0, npairs, pair_body, 0)

    @pl.when(npairs > 0)
    def _():
        pltpu.make_async_copy(ob0, o_slice(0), so0).wait()
        pltpu.make_async_copy(ob1, o_slice(0), so1).wait()

    @pl.when(num_chunks % 2 == 1)
    def _():
        c = num_chunks - 1
        pltpu.sync_copy(x_slice(c), xb0.at[pl.ds(0, XW)])
        compute(xb0, ob0)
        pltpu.sync_copy(ob0, o_slice(c))


def kernel(x, tables):
    n = x.shape[0]
    n32 = ((n + CHUNK - 1) // CHUNK) * CHUNK
    if n32 != n:
        x = jnp.pad(x, ((0, n32 - n), (0, 0)))
    rows_per_worker = ((n32 + NWORKERS * CHUNK - 1) // (NWORKERS * CHUNK)) * CHUNK
    last_rows = n32 - (NWORKERS - 1) * rows_per_worker
    assert last_rows >= 0

    x_flat = x.astype(jnp.int32).reshape(-1)
    tab_flat = _pack_table(tables.reshape(NUM_F * VOCAB, HIDDEN))

    mesh = plsc.VectorSubcoreMesh(
        core_axis_name="c", subcore_axis_name="s", num_cores=2, num_subcores=16
    )
    run = pl.kernel(
        functools.partial(_body, rows_per_worker, last_rows),
        out_type=jax.ShapeDtypeStruct((n32, HIDDEN), jnp.float32),
        mesh=mesh,
        compiler_params=pltpu.CompilerParams(needs_layout_passes=False),
        scratch_types=[
            pltpu.VMEM((TAB_WORDS // 2,), jnp.int32),
            pltpu.VMEM((XW + 16,), jnp.int32),
            pltpu.VMEM((XW + 16,), jnp.int32),
            pltpu.VMEM((CHUNK, HIDDEN), jnp.float32),
            pltpu.VMEM((CHUNK, HIDDEN), jnp.float32),
            pltpu.SemaphoreType.DMA,
            pltpu.SemaphoreType.DMA,
            pltpu.SemaphoreType.DMA,
            pltpu.SemaphoreType.DMA,
        ],
    )
    out = run(x_flat, tab_flat)
    return out[:n] if n32 != n else out
